# baseline (speedup 1.0000x reference)
import functools

import jax
import jax.numpy as jnp
from jax import lax
from jax.experimental import pallas as pl
from jax.experimental.pallas import tpu as pltpu
from jax.experimental.pallas import tpu_sc as plsc

N = 10000
CIN = 128
COUT = 128
KDIM = 27

_NPAD = 10240
_B = KDIM * _NPAD
_NW = 32
_PER_W = _B // _NW
_WIN = 120
_STEPS = _PER_W // _WIN
_RING = 6

_MBLK = 1000
_PK = CIN // 2


def _sc_gather(data, idx):
    mesh = plsc.VectorSubcoreMesh(core_axis_name="c", subcore_axis_name="s")

    @functools.partial(
        pl.kernel,
        out_type=jax.ShapeDtypeStruct((_B, _PK), jnp.int32),
        mesh=mesh,
        compiler_params=pltpu.CompilerParams(use_tc_tiling_on_sc=False),
        scratch_types=[
            pltpu.VMEM((_PER_W,), jnp.int32),
            pltpu.VMEM((_RING, _WIN, _PK), jnp.int32),
            pltpu.SemaphoreType.DMA((_RING,)),
            pltpu.SemaphoreType.DMA((_RING,)),
            pltpu.SemaphoreType.DMA,
        ],
    )
    def gather_kernel(data_hbm, idx_hbm, out_hbm, idx_v, rows_v, gsem, wsem, isem):
        wid = lax.axis_index("c") * 16 + lax.axis_index("s")
        base = wid * _PER_W
        pltpu.async_copy(idx_hbm.at[pl.ds(base, _PER_W)], idx_v, isem).wait()

        def g_start(s, b):
            pltpu.make_async_copy(
                data_hbm.at[idx_v.at[pl.ds(s * _WIN, _WIN)]],
                rows_v.at[b], gsem.at[b]).start()

        def g_wait(b):
            pltpu.make_async_copy(
                data_hbm.at[idx_v.at[pl.ds(0, _WIN)]],
                rows_v.at[b], gsem.at[b]).wait()

        def w_start(s, b):
            pltpu.make_async_copy(
                rows_v.at[b], out_hbm.at[pl.ds(base + s * _WIN, _WIN)],
                wsem.at[b]).start()

        def w_wait(b):
            pltpu.make_async_copy(
                rows_v.at[b], out_hbm.at[pl.ds(base, _WIN)],
                wsem.at[b]).wait()

        for b in range(_RING):
            g_start(b, b)

        @pl.loop(0, _STEPS - _RING, step=_RING)
        def _(s):
            for b in range(_RING):
                g_wait(b)
                w_start(s + b, b)
            for b in range(_RING):
                w_wait(b)
                g_start(s + _RING + b, b)

        s_last = _STEPS - _RING
        for b in range(_RING):
            g_wait(b)
            w_start(s_last + b, b)
        for b in range(_RING):
            w_wait(b)

    return gather_kernel(data, idx)


def _gemm_body(buf_ref, w_ref, out_ref):
    acc = jnp.zeros_like(out_ref)
    for k in range(KDIM):
        a = buf_ref[k]
        w = w_ref[k].astype(jnp.bfloat16)
        acc += jnp.dot(a, w, preferred_element_type=jnp.float32)
    out_ref[...] = acc


def _tc_gemm(buffer, weights):
    n_m = N // _MBLK
    buffer3 = jax.lax.bitcast_convert_type(buffer, jnp.bfloat16)
    buffer3 = buffer3.reshape(KDIM, _NPAD, CIN)
    return pl.pallas_call(
        _gemm_body,
        grid=(n_m,),
        in_specs=[
            pl.BlockSpec((KDIM, _MBLK, CIN), lambda m: (0, m, 0)),
            pl.BlockSpec((KDIM, CIN, COUT), lambda m: (0, 0, 0)),
        ],
        out_specs=pl.BlockSpec((_MBLK, COUT), lambda m: (m, 0)),
        out_shape=jax.ShapeDtypeStruct((N, COUT), jnp.float32),
    )(buffer3, weights)


def kernel(data, weights, neigh):
    idx = jnp.pad(neigh.T, ((0, 0), (0, _NPAD - N)))
    idx = idx.reshape(_B)
    data_pk = jax.lax.bitcast_convert_type(
        data.astype(jnp.bfloat16).reshape(N, _PK, 2), jnp.int32)
    buffer = _sc_gather(data_pk, idx)
    return _tc_gemm(buffer, weights)


# speedup vs baseline: 1.1856x; 1.1856x over previous
import functools

import jax
import jax.numpy as jnp
from jax import lax
from jax.experimental import pallas as pl
from jax.experimental.pallas import tpu as pltpu
from jax.experimental.pallas import tpu_sc as plsc

N = 10000
CIN = 128
COUT = 128
KDIM = 27

_NPAD = 10240
_KP = KDIM + 1
_NW = 32
_ROWS_W = _NPAD // _NW
_G = 4
_WIN = _G * _KP
_NWIN = _ROWS_W // _G
_RING = 4

_MBLK = 1000


def _tc_pgemm(data, weights):
    def body(d_ref, w_ref, p_ref):
        d = d_ref[...].astype(jnp.bfloat16)
        for k in range(KDIM):
            w = w_ref[k].astype(jnp.bfloat16)
            p_ref[k] = jnp.dot(d, w, preferred_element_type=jnp.float32)

    return pl.pallas_call(
        body,
        grid=(N // _MBLK,),
        in_specs=[
            pl.BlockSpec((_MBLK, CIN), lambda m: (m, 0)),
            pl.BlockSpec((KDIM, CIN, COUT), lambda m: (0, 0, 0)),
        ],
        out_specs=pl.BlockSpec((KDIM, _MBLK, COUT), lambda m: (0, m, 0)),
        out_shape=jax.ShapeDtypeStruct((KDIM, N, COUT), jnp.float32),
    )(data, weights)


def _sc_gather_acc(table, idx):
    mesh = plsc.VectorSubcoreMesh(core_axis_name="c", subcore_axis_name="s")

    @functools.partial(
        pl.kernel,
        out_type=jax.ShapeDtypeStruct((_NPAD, COUT), jnp.float32),
        mesh=mesh,
        scratch_types=[
            pltpu.VMEM((_ROWS_W * _KP,), jnp.int32),
            pltpu.VMEM((_RING, _WIN, COUT), jnp.float32),
            pltpu.VMEM((_ROWS_W, COUT), jnp.float32),
            pltpu.SemaphoreType.DMA((_RING,)),
            pltpu.SemaphoreType.DMA,
        ],
    )
    def acc_kernel(table_hbm, idx_hbm, out_hbm, idx_v, rows_v, out_v, gsem, osem):
        wid = lax.axis_index("c") * 16 + lax.axis_index("s")
        pltpu.async_copy(
            idx_hbm.at[pl.ds(wid * _ROWS_W * _KP, _ROWS_W * _KP)], idx_v, osem
        ).wait()

        def g_start(w, b):
            pltpu.make_async_copy(
                table_hbm.at[idx_v.at[pl.ds(w * _WIN, _WIN)]],
                rows_v.at[b], gsem.at[b]).start()

        def g_wait(b):
            pltpu.make_async_copy(
                table_hbm.at[idx_v.at[pl.ds(0, _WIN)]],
                rows_v.at[b], gsem.at[b]).wait()

        def accumulate(w, b):
            @pl.loop(0, _G)
            def _(r):
                row0 = r * _KP
                orow = w * _G + r
                for c in range(0, COUT, 16):
                    acc = rows_v[b, row0, pl.ds(c, 16)]
                    for k in range(1, KDIM):
                        acc += rows_v[b, row0 + k, pl.ds(c, 16)]
                    out_v[orow, pl.ds(c, 16)] = acc

        for b in range(_RING):
            g_start(b, b)

        @pl.loop(0, _NWIN - _RING, step=_RING)
        def _(w):
            for b in range(_RING):
                g_wait(b)
                accumulate(w + b, b)
                g_start(w + _RING + b, b)

        w_last = _NWIN - _RING
        for b in range(_RING):
            g_wait(b)
            accumulate(w_last + b, b)

        pltpu.async_copy(
            out_v, out_hbm.at[pl.ds(wid * _ROWS_W, _ROWS_W)], osem
        ).wait()

    return acc_kernel(table, idx)


def kernel(data, weights, neigh):
    idx = neigh + (N * jnp.arange(KDIM, dtype=jnp.int32))[None, :]
    idx = jnp.pad(idx, ((0, _NPAD - N), (0, _KP - KDIM)))
    idx = idx.reshape(_NPAD * _KP)
    table = _tc_pgemm(data, weights).reshape(KDIM * N, COUT)
    out = _sc_gather_acc(table, idx)
    return out[:N]


# speedup vs baseline: 1.1871x; 1.0012x over previous
import functools

import jax
import jax.numpy as jnp
from jax import lax
from jax.experimental import pallas as pl
from jax.experimental.pallas import tpu as pltpu
from jax.experimental.pallas import tpu_sc as plsc

N = 10000
CIN = 128
COUT = 128
KDIM = 27

_NPAD = 10240
_KP = KDIM + 1
_NW = 32
_ROWS_W = _NPAD // _NW
_G = 4
_WIN = _G * _KP
_NWIN = _ROWS_W // _G
_RING = 4

_MBLK = 1000


def _tc_pgemm(data, weights):
    def body(d_ref, w_ref, p_ref):
        d = d_ref[...].astype(jnp.bfloat16)
        for k in range(KDIM):
            w = w_ref[k].astype(jnp.bfloat16)
            p_ref[k] = jnp.dot(d, w, preferred_element_type=jnp.float32)

    return pl.pallas_call(
        body,
        grid=(N // _MBLK,),
        in_specs=[
            pl.BlockSpec((_MBLK, CIN), lambda m: (m, 0)),
            pl.BlockSpec((KDIM, CIN, COUT), lambda m: (0, 0, 0)),
        ],
        out_specs=pl.BlockSpec((KDIM, _MBLK, COUT), lambda m: (0, m, 0)),
        out_shape=jax.ShapeDtypeStruct((KDIM, N, COUT), jnp.float32),
    )(data, weights)


def _sc_gather_acc(table, idx):
    mesh = plsc.VectorSubcoreMesh(core_axis_name="c", subcore_axis_name="s")

    @functools.partial(
        pl.kernel,
        out_type=jax.ShapeDtypeStruct((_NPAD, COUT), jnp.float32),
        mesh=mesh,
        scratch_types=[
            pltpu.VMEM((_ROWS_W * _KP,), jnp.int32),
            pltpu.VMEM((_RING, _WIN, COUT), jnp.float32),
            pltpu.VMEM((_ROWS_W, COUT), jnp.float32),
            pltpu.SemaphoreType.DMA((_RING,)),
            pltpu.SemaphoreType.DMA,
        ],
    )
    def acc_kernel(table_hbm, idx_hbm, out_hbm, idx_v, rows_v, out_v, gsem, osem):
        wid = lax.axis_index("c") * 16 + lax.axis_index("s")
        pltpu.async_copy(
            idx_hbm.at[pl.ds(wid * _ROWS_W * _KP, _ROWS_W * _KP)], idx_v, osem
        ).wait()

        def g_start(w, b):
            pltpu.make_async_copy(
                table_hbm.at[idx_v.at[pl.ds(w * _WIN, _WIN)]],
                rows_v.at[b], gsem.at[b]).start()

        def g_wait(b):
            pltpu.make_async_copy(
                table_hbm.at[idx_v.at[pl.ds(0, _WIN)]],
                rows_v.at[b], gsem.at[b]).wait()

        def accumulate(w, b):
            @pl.loop(0, _G)
            def _(r):
                row0 = r * _KP
                orow = w * _G + r
                for c in range(0, COUT, 32):
                    ca = [None] * 4
                    cb = [None] * 4
                    for k in range(KDIM):
                        la = rows_v[b, row0 + k, pl.ds(c, 16)]
                        lb = rows_v[b, row0 + k, pl.ds(c + 16, 16)]
                        j = k % 4
                        ca[j] = la if k < 4 else ca[j] + la
                        cb[j] = lb if k < 4 else cb[j] + lb
                    out_v[orow, pl.ds(c, 16)] = (ca[0] + ca[1]) + (ca[2] + ca[3])
                    out_v[orow, pl.ds(c + 16, 16)] = (cb[0] + cb[1]) + (cb[2] + cb[3])

        for b in range(_RING):
            g_start(b, b)

        @pl.loop(0, _NWIN - _RING, step=_RING)
        def _(w):
            for b in range(_RING):
                g_wait(b)
                accumulate(w + b, b)
                g_start(w + _RING + b, b)

        w_last = _NWIN - _RING
        for b in range(_RING):
            g_wait(b)
            accumulate(w_last + b, b)

        pltpu.async_copy(
            out_v, out_hbm.at[pl.ds(wid * _ROWS_W, _ROWS_W)], osem
        ).wait()

    return acc_kernel(table, idx)


def kernel(data, weights, neigh):
    idx = neigh + (N * jnp.arange(KDIM, dtype=jnp.int32))[None, :]
    idx = jnp.pad(idx, ((0, _NPAD - N), (0, _KP - KDIM)))
    idx = idx.reshape(_NPAD * _KP)
    table = _tc_pgemm(data, weights).reshape(KDIM * N, COUT)
    out = _sc_gather_acc(table, idx)
    return out[:N]


# speedup vs baseline: 2.3030x; 1.9400x over previous
import dataclasses
import functools

import jax
import jax.numpy as jnp
from jax import lax
from jax.experimental import pallas as pl
from jax.experimental.pallas import tpu as pltpu
from jax.experimental.pallas import tpu_sc as plsc

N = 10000
CIN = 128
COUT = 128
KDIM = 27

_NPAD = 10240
_B = KDIM * _NPAD
_NW = 32
_PER_W = _B // _NW
_WIN = 128
_NFULL = 67
_TAIL = _PER_W - _NFULL * _WIN
_RING = 4

_MBLK = 1024


def _sc_gather_pack(data, idx):
    mesh = plsc.VectorSubcoreMesh(core_axis_name="c", subcore_axis_name="s")
    cp = pltpu.CompilerParams()
    if "needs_layout_passes" in pltpu.CompilerParams.__dataclass_fields__:
        cp = dataclasses.replace(cp, needs_layout_passes=False)

    @functools.partial(
        pl.kernel,
        out_type=jax.ShapeDtypeStruct((_B // 2, CIN), jnp.int32),
        mesh=mesh,
        compiler_params=cp,
        scratch_types=[
            pltpu.VMEM((_PER_W,), jnp.int32),
            pltpu.VMEM((_RING, _WIN, CIN), jnp.float32),
            pltpu.VMEM((_RING, _WIN // 2, CIN), jnp.int32),
            pltpu.SemaphoreType.DMA((_RING,)),
            pltpu.SemaphoreType.DMA((_RING,)),
            pltpu.SemaphoreType.DMA,
        ],
    )
    def gather_kernel(data_hbm, idx_hbm, out_hbm, idx_v, rows_v, pk_v,
                      gsem, wsem, isem):
        wid = lax.axis_index("c") * 16 + lax.axis_index("s")
        base = wid * _PER_W
        pbase = wid * (_PER_W // 2)
        pltpu.async_copy(idx_hbm.at[pl.ds(base, _PER_W)], idx_v, isem).wait()

        def g_start(w, b, n):
            pltpu.make_async_copy(
                data_hbm.at[idx_v.at[pl.ds(w * _WIN, n)]],
                rows_v.at[b, pl.ds(0, n)], gsem.at[b]).start()

        def g_wait(b, n):
            pltpu.make_async_copy(
                data_hbm.at[idx_v.at[pl.ds(0, n)]],
                rows_v.at[b, pl.ds(0, n)], gsem.at[b]).wait()

        def w_start(w, b, n):
            pltpu.make_async_copy(
                pk_v.at[b, pl.ds(0, n // 2)],
                out_hbm.at[pl.ds(pbase + w * (_WIN // 2), n // 2)],
                wsem.at[b]).start()

        def w_wait(b, n):
            pltpu.make_async_copy(
                pk_v.at[b, pl.ds(0, n // 2)],
                out_hbm.at[pl.ds(pbase, n // 2)],
                wsem.at[b]).wait()

        half = jnp.uint32(0x8000)
        mask = jnp.uint32(0xFFFF0000)

        def pack(b, npairs):
            @pl.loop(0, npairs)
            def _(r2):
                for j in range(0, CIN, 16):
                    lo32 = plsc.bitcast(rows_v[b, 2 * r2, pl.ds(j, 16)],
                                        jnp.uint32)
                    hi32 = plsc.bitcast(rows_v[b, 2 * r2 + 1, pl.ds(j, 16)],
                                        jnp.uint32)
                    lo = lax.shift_right_logical(lo32 + half, jnp.uint32(16))
                    hi = (hi32 + half) & mask
                    pk_v[b, r2, pl.ds(j, 16)] = plsc.bitcast(lo | hi, jnp.int32)

        for b in range(_RING):
            g_start(b, b, _WIN)

        for b in range(_RING):
            g_wait(b, _WIN)
            pack(b, _WIN // 2)
            w_start(b, b, _WIN)
            g_start(_RING + b, b, _WIN)

        @pl.loop(_RING, _NFULL - 2 * _RING + 1, step=_RING)
        def _(w):
            for b in range(_RING):
                g_wait(b, _WIN)
                w_wait(b, _WIN)
                pack(b, _WIN // 2)
                w_start(w + b, b, _WIN)
                g_start(w + _RING + b, b, _WIN)

        for b, w in enumerate(range(60, 64)):
            g_wait(b, _WIN)
            w_wait(b, _WIN)
            pack(b, _WIN // 2)
            w_start(w, b, _WIN)
            g_start(w + _RING, b, _WIN if w + _RING < _NFULL else _TAIL)

        for b, w in enumerate(range(64, 68)):
            n = _WIN if w < _NFULL else _TAIL
            g_wait(b, n)
            w_wait(b, _WIN)
            pack(b, n // 2)
            w_start(w, b, n)

        for b, w in enumerate(range(64, 68)):
            w_wait(b, _WIN if w < _NFULL else _TAIL)

    return gather_kernel(data, idx)


def _gemm_body(buf_ref, w_ref, out_ref):
    acc = jnp.zeros_like(out_ref)
    for k in range(KDIM):
        a = pltpu.bitcast(buf_ref[k], jnp.bfloat16)
        w = w_ref[k].astype(jnp.bfloat16)
        acc += jnp.dot(a, w, preferred_element_type=jnp.float32)
    out_ref[...] = acc


def _tc_gemm(packed, weights):
    n_m = _NPAD // _MBLK
    buffer3 = packed.reshape(KDIM, _NPAD // 2, CIN)
    return pl.pallas_call(
        _gemm_body,
        grid=(n_m,),
        in_specs=[
            pl.BlockSpec((KDIM, _MBLK // 2, CIN), lambda m: (0, m, 0)),
            pl.BlockSpec((KDIM, CIN, COUT), lambda m: (0, 0, 0)),
        ],
        out_specs=pl.BlockSpec((_MBLK, COUT), lambda m: (m, 0)),
        out_shape=jax.ShapeDtypeStruct((_NPAD, COUT), jnp.float32),
    )(buffer3, weights)


def kernel(data, weights, neigh):
    idx = jnp.pad(neigh.T, ((0, 0), (0, _NPAD - N)))
    idx = idx.reshape(_B)
    packed = _sc_gather_pack(data, idx)
    return _tc_gemm(packed, weights)[:N]


# speedup vs baseline: 2.3729x; 1.0304x over previous
import dataclasses
import functools

import jax
import jax.numpy as jnp
from jax import lax
from jax.experimental import pallas as pl
from jax.experimental.pallas import tpu as pltpu
from jax.experimental.pallas import tpu_sc as plsc

N = 10000
CIN = 128
COUT = 128
KDIM = 27

_NPAD = 10240
_B = KDIM * _NPAD
_NW = 32
_PER_W = _B // _NW
_WIN = 128
_NFULL = 67
_TAIL = _PER_W - _NFULL * _WIN
_RING = 4

_MBLK = 1024


def _sc_gather_pack(data, idx):
    mesh = plsc.VectorSubcoreMesh(core_axis_name="c", subcore_axis_name="s")
    cp = pltpu.CompilerParams()
    if "needs_layout_passes" in pltpu.CompilerParams.__dataclass_fields__:
        cp = dataclasses.replace(cp, needs_layout_passes=False)

    @functools.partial(
        pl.kernel,
        out_type=jax.ShapeDtypeStruct((_B // 2, CIN), jnp.int32),
        mesh=mesh,
        compiler_params=cp,
        scratch_types=[
            pltpu.VMEM((_PER_W,), jnp.int32),
            pltpu.VMEM((_RING, _WIN, CIN), jnp.float32),
            pltpu.VMEM((_RING, _WIN // 2, CIN), jnp.int32),
            pltpu.SemaphoreType.DMA((_RING,)),
            pltpu.SemaphoreType.DMA((_RING,)),
            pltpu.SemaphoreType.DMA,
        ],
    )
    def gather_kernel(data_hbm, idx_hbm, out_hbm, idx_v, rows_v, pk_v,
                      gsem, wsem, isem):
        wid = lax.axis_index("c") * 16 + lax.axis_index("s")
        base = wid * _PER_W
        pbase = wid * (_PER_W // 2)
        pltpu.async_copy(idx_hbm.at[pl.ds(base, _PER_W)], idx_v, isem).wait()

        def g_start(w, b, n):
            pltpu.make_async_copy(
                data_hbm.at[idx_v.at[pl.ds(w * _WIN, n)]],
                rows_v.at[b, pl.ds(0, n)], gsem.at[b]).start()

        def g_wait(b, n):
            pltpu.make_async_copy(
                data_hbm.at[idx_v.at[pl.ds(0, n)]],
                rows_v.at[b, pl.ds(0, n)], gsem.at[b]).wait()

        def w_start(w, b, n):
            pltpu.make_async_copy(
                pk_v.at[b, pl.ds(0, n // 2)],
                out_hbm.at[pl.ds(pbase + w * (_WIN // 2), n // 2)],
                wsem.at[b]).start()

        def w_wait(b, n):
            pltpu.make_async_copy(
                pk_v.at[b, pl.ds(0, n // 2)],
                out_hbm.at[pl.ds(pbase, n // 2)],
                wsem.at[b]).wait()

        def pack(b, npairs):
            @pl.loop(0, npairs)
            def _(r2):
                for j in range(0, CIN, 16):
                    pair = plsc.pack(rows_v[b, 2 * r2, pl.ds(j, 16)],
                                     rows_v[b, 2 * r2 + 1, pl.ds(j, 16)],
                                     format=plsc.PackFormat.INTERLEAVED)
                    pk_v[b, r2, pl.ds(j, 16)] = plsc.bitcast(pair, jnp.int32)

        for b in range(_RING):
            g_start(b, b, _WIN)

        for b in range(_RING):
            g_wait(b, _WIN)
            pack(b, _WIN // 2)
            w_start(b, b, _WIN)
            g_start(_RING + b, b, _WIN)

        @pl.loop(_RING, _NFULL - 2 * _RING + 1, step=_RING)
        def _(w):
            for b in range(_RING):
                g_wait(b, _WIN)
                w_wait(b, _WIN)
                pack(b, _WIN // 2)
                w_start(w + b, b, _WIN)
                g_start(w + _RING + b, b, _WIN)

        for b, w in enumerate(range(60, 64)):
            g_wait(b, _WIN)
            w_wait(b, _WIN)
            pack(b, _WIN // 2)
            w_start(w, b, _WIN)
            g_start(w + _RING, b, _WIN if w + _RING < _NFULL else _TAIL)

        for b, w in enumerate(range(64, 68)):
            n = _WIN if w < _NFULL else _TAIL
            g_wait(b, n)
            w_wait(b, _WIN)
            pack(b, n // 2)
            w_start(w, b, n)

        for b, w in enumerate(range(64, 68)):
            w_wait(b, _WIN if w < _NFULL else _TAIL)

    return gather_kernel(data, idx)


def _gemm_body(buf_ref, w_ref, out_ref):
    acc = jnp.zeros_like(out_ref)
    for k in range(KDIM):
        a = pltpu.bitcast(buf_ref[k], jnp.bfloat16)
        w = w_ref[k].astype(jnp.bfloat16)
        acc += jnp.dot(a, w, preferred_element_type=jnp.float32)
    out_ref[...] = acc


def _tc_gemm(packed, weights):
    n_m = _NPAD // _MBLK
    buffer3 = packed.reshape(KDIM, _NPAD // 2, CIN)
    return pl.pallas_call(
        _gemm_body,
        grid=(n_m,),
        in_specs=[
            pl.BlockSpec((KDIM, _MBLK // 2, CIN), lambda m: (0, m, 0)),
            pl.BlockSpec((KDIM, CIN, COUT), lambda m: (0, 0, 0)),
        ],
        out_specs=pl.BlockSpec((_MBLK, COUT), lambda m: (m, 0)),
        out_shape=jax.ShapeDtypeStruct((_NPAD, COUT), jnp.float32),
    )(buffer3, weights)


def kernel(data, weights, neigh):
    idx = jnp.pad(neigh.T, ((0, 0), (0, _NPAD - N)))
    idx = idx.reshape(_B)
    packed = _sc_gather_pack(data, idx)
    return _tc_gemm(packed, weights)[:N]


# speedup vs baseline: 2.4029x; 1.0126x over previous
import dataclasses
import functools

import jax
import jax.numpy as jnp
from jax import lax
from jax.experimental import pallas as pl
from jax.experimental.pallas import tpu as pltpu
from jax.experimental.pallas import tpu_sc as plsc

N = 10000
CIN = 128
COUT = 128
KDIM = 27

_NPAD = 10240
_B = KDIM * _NPAD
_NW = 32
_PER_W = _B // _NW
_WIN = 128
_NFULL = 67
_TAIL = _PER_W - _NFULL * _WIN
_RING = 4

_MBLK = 1024


def _sc_gather_pack(data, idx):
    mesh = plsc.VectorSubcoreMesh(core_axis_name="c", subcore_axis_name="s")
    cp = pltpu.CompilerParams()
    if "needs_layout_passes" in pltpu.CompilerParams.__dataclass_fields__:
        cp = dataclasses.replace(cp, needs_layout_passes=False)

    @functools.partial(
        pl.kernel,
        out_type=jax.ShapeDtypeStruct((_B // 2, CIN), jnp.int32),
        mesh=mesh,
        compiler_params=cp,
        scratch_types=[
            pltpu.VMEM((_PER_W,), jnp.int32),
            pltpu.VMEM((_RING, _WIN, CIN), jnp.float32),
            pltpu.VMEM((2, _WIN, CIN), jnp.int32),
            pltpu.SemaphoreType.DMA((_RING,)),
            pltpu.SemaphoreType.DMA((2,)),
            pltpu.SemaphoreType.DMA,
        ],
    )
    def gather_kernel(data_hbm, idx_hbm, out_hbm, idx_v, rows_v, pk_v,
                      gsem, wsem, isem):
        wid = lax.axis_index("c") * 16 + lax.axis_index("s")
        base = wid * _PER_W
        pbase = wid * (_PER_W // 2)
        pltpu.async_copy(idx_hbm.at[pl.ds(base, _PER_W)], idx_v, isem).wait()

        def g_start(w, b, n):
            pltpu.make_async_copy(
                data_hbm.at[idx_v.at[pl.ds(w * _WIN, n)]],
                rows_v.at[b, pl.ds(0, n)], gsem.at[b]).start()

        def g_wait(b, n):
            pltpu.make_async_copy(
                data_hbm.at[idx_v.at[pl.ds(0, n)]],
                rows_v.at[b, pl.ds(0, n)], gsem.at[b]).wait()

        def w_start(w, p, n):
            pltpu.make_async_copy(
                pk_v.at[p, pl.ds(0, n)],
                out_hbm.at[pl.ds(pbase + w * (_WIN // 2), n)],
                wsem.at[p]).start()

        def w_wait(p, n):
            pltpu.make_async_copy(
                pk_v.at[p, pl.ds(0, n)],
                out_hbm.at[pl.ds(pbase, n)],
                wsem.at[p]).wait()

        def pack(b, p, half, npairs):
            @pl.loop(0, npairs)
            def _(r2):
                for j in range(0, CIN, 16):
                    pair = plsc.pack(rows_v[b, 2 * r2, pl.ds(j, 16)],
                                     rows_v[b, 2 * r2 + 1, pl.ds(j, 16)],
                                     format=plsc.PackFormat.INTERLEAVED)
                    pk_v[p, half * (_WIN // 2) + r2, pl.ds(j, 16)] = (
                        plsc.bitcast(pair, jnp.int32))

        for b in range(_RING):
            g_start(b, b, _WIN)

        for b in range(_RING):
            g_wait(b, _WIN)
            pack(b, b // 2, b % 2, _WIN // 2)
            if b % 2 == 1:
                w_start(b - 1, b // 2, _WIN)
            g_start(_RING + b, b, _WIN)

        @pl.loop(_RING, _NFULL - 2 * _RING + 1, step=_RING)
        def _(w):
            for b in range(_RING):
                g_wait(b, _WIN)
                if b % 2 == 0:
                    w_wait(b // 2, _WIN)
                pack(b, b // 2, b % 2, _WIN // 2)
                if b % 2 == 1:
                    w_start(w + b - 1, b // 2, _WIN)
                g_start(w + _RING + b, b, _WIN)

        for b, w in enumerate(range(60, 64)):
            g_wait(b, _WIN)
            if b % 2 == 0:
                w_wait(b // 2, _WIN)
            pack(b, b // 2, b % 2, _WIN // 2)
            if b % 2 == 1:
                w_start(w - 1, b // 2, _WIN)
            g_start(w + _RING, b, _WIN if w + _RING < _NFULL else _TAIL)

        for b, w in enumerate(range(64, 68)):
            n = _WIN if w < _NFULL else _TAIL
            g_wait(b, n)
            if b % 2 == 0:
                w_wait(b // 2, _WIN)
            pack(b, b // 2, b % 2, n // 2)
            if b % 2 == 1:
                w_start(w - 1, b // 2, (_WIN + n) // 2)

        w_wait(0, _WIN)
        w_wait(1, (_WIN + _TAIL) // 2)

    return gather_kernel(data, idx)


def _gemm_body(buf_ref, w_ref, out_ref):
    acc = jnp.zeros_like(out_ref)
    for k in range(KDIM):
        a = pltpu.bitcast(buf_ref[k], jnp.bfloat16)
        w = w_ref[k].astype(jnp.bfloat16)
        acc += jnp.dot(a, w, preferred_element_type=jnp.float32)
    out_ref[...] = acc


def _tc_gemm(packed, weights):
    n_m = _NPAD // _MBLK
    buffer3 = packed.reshape(KDIM, _NPAD // 2, CIN)
    return pl.pallas_call(
        _gemm_body,
        grid=(n_m,),
        in_specs=[
            pl.BlockSpec((KDIM, _MBLK // 2, CIN), lambda m: (0, m, 0)),
            pl.BlockSpec((KDIM, CIN, COUT), lambda m: (0, 0, 0)),
        ],
        out_specs=pl.BlockSpec((_MBLK, COUT), lambda m: (m, 0)),
        out_shape=jax.ShapeDtypeStruct((_NPAD, COUT), jnp.float32),
    )(buffer3, weights)


def kernel(data, weights, neigh):
    idx = jnp.pad(neigh.T, ((0, 0), (0, _NPAD - N)))
    idx = idx.reshape(_B)
    packed = _sc_gather_pack(data, idx)
    return _tc_gemm(packed, weights)[:N]
